# Initial kernel scaffold; baseline (speedup 1.0000x reference)
import functools

import jax
import jax.numpy as jnp
from jax import lax
from jax.experimental import pallas as pl
from jax.experimental.pallas import tpu as pltpu
from jax.experimental.pallas import tpu_sc as plsc

_B = 4096
_L = 200
_DIM = 64
_NTOT = _B * _L
_NC = 2
_NS = 16
_NW = _NC * _NS
_PER_W = _NTOT // _NW
_CHUNK = 512
_NCHUNK = _PER_W // _CHUNK
_SUB = _CHUNK // 128
_IDX_ROWS = _NTOT // 128
_ROWS_PER_W = _PER_W // 128


@functools.partial(
    pl.kernel,
    mesh=plsc.VectorSubcoreMesh(core_axis_name="c", subcore_axis_name="s"),
    out_type=jax.ShapeDtypeStruct((_NTOT, _DIM), jnp.float32),
    scratch_types=[
        pltpu.VMEM((_SUB, 128), jnp.int32),
        pltpu.VMEM((_CHUNK, _DIM), jnp.float32),
        pltpu.SemaphoreType.DMA,
    ],
)
def _lookup(idx_hbm, table_hbm, out_hbm, idx_v, rows_v, sem):
    wid = lax.axis_index("s") * _NC + lax.axis_index("c")
    idx_row0 = wid * _ROWS_PER_W
    out_row0 = wid * _PER_W

    def body(c, carry):
        pltpu.sync_copy(idx_hbm.at[pl.ds(idx_row0 + c * _SUB, _SUB)], idx_v)
        copies = []
        for j in range(_SUB):
            copies.append(
                pltpu.async_copy(
                    table_hbm.at[idx_v.at[j]],
                    rows_v.at[pl.ds(j * 128, 128)],
                    sem,
                )
            )
        for cp in copies:
            cp.wait()
        pltpu.sync_copy(rows_v, out_hbm.at[pl.ds(out_row0 + c * _CHUNK, _CHUNK)])
        return carry

    lax.fori_loop(0, _NCHUNK, body, 0)


def kernel(idxes, pe):
    idx2d = idxes.astype(jnp.int32).reshape(_IDX_ROWS, 128)
    out = _lookup(idx2d, pe)
    return out.reshape(_B, _L, _DIM)


# speedup vs baseline: 4.6457x; 4.6457x over previous
import functools

import jax
import jax.numpy as jnp
from jax import lax
from jax.experimental import pallas as pl
from jax.experimental.pallas import tpu as pltpu
from jax.experimental.pallas import tpu_sc as plsc

_B = 4096
_L = 200
_DIM = 64
_NTOT = _B * _L
_NC = 2
_NS = 16
_NW = _NC * _NS
_PER_W = _NTOT // _NW
_CHUNK = 512
_NCHUNK = _PER_W // _CHUNK
_SUB = _CHUNK // 128
_IDX_ROWS = _NTOT // 128
_ROWS_PER_W = _PER_W // 128


@functools.partial(
    pl.kernel,
    mesh=plsc.VectorSubcoreMesh(core_axis_name="c", subcore_axis_name="s"),
    out_type=jax.ShapeDtypeStruct((_NTOT, _DIM), jnp.float32),
    scratch_types=[
        pltpu.VMEM((_SUB, 128), jnp.int32),
        pltpu.VMEM((_CHUNK, _DIM), jnp.float32),
        pltpu.SemaphoreType.DMA,
    ],
    compiler_params=pltpu.CompilerParams(use_tc_tiling_on_sc=False),
)
def _lookup(idx_hbm, table_hbm, out_hbm, idx_v, rows_v, sem):
    wid = lax.axis_index("s") * _NC + lax.axis_index("c")
    idx_row0 = wid * _ROWS_PER_W
    out_row0 = wid * _PER_W

    def body(c, carry):
        pltpu.sync_copy(idx_hbm.at[pl.ds(idx_row0 + c * _SUB, _SUB)], idx_v)
        copies = []
        for j in range(_SUB):
            copies.append(
                pltpu.async_copy(
                    table_hbm.at[idx_v.at[j]],
                    rows_v.at[pl.ds(j * 128, 128)],
                    sem,
                )
            )
        for cp in copies:
            cp.wait()
        pltpu.sync_copy(rows_v, out_hbm.at[pl.ds(out_row0 + c * _CHUNK, _CHUNK)])
        return carry

    lax.fori_loop(0, _NCHUNK, body, 0)


def kernel(idxes, pe):
    idx2d = idxes.astype(jnp.int32).reshape(_IDX_ROWS, 128)
    out = _lookup(idx2d, pe)
    return out.reshape(_B, _L, _DIM)


# speedup vs baseline: 4.9725x; 1.0704x over previous
import functools

import jax
import jax.numpy as jnp
from jax import lax
from jax.experimental import pallas as pl
from jax.experimental.pallas import tpu as pltpu
from jax.experimental.pallas import tpu_sc as plsc

_B = 4096
_L = 200
_DIM = 64
_NTOT = _B * _L
_NC = 2
_NS = 16
_NW = _NC * _NS
_PER_W = _NTOT // _NW
_CHUNK = 512
_NCHUNK = _PER_W // _CHUNK
_SUB = _CHUNK // 128
_IDX_ROWS = _NTOT // 128
_ROWS_PER_W = _PER_W // 128


@functools.partial(
    pl.kernel,
    mesh=plsc.VectorSubcoreMesh(core_axis_name="c", subcore_axis_name="s"),
    out_type=jax.ShapeDtypeStruct((_NTOT, _DIM), jnp.float32),
    scratch_types=[
        pltpu.VMEM((_ROWS_PER_W, 128), jnp.int32),
        pltpu.VMEM((_CHUNK, _DIM), jnp.float32),
        pltpu.VMEM((_CHUNK, _DIM), jnp.float32),
        pltpu.SemaphoreType.DMA,
        pltpu.SemaphoreType.DMA,
        pltpu.SemaphoreType.DMA,
        pltpu.SemaphoreType.DMA,
    ],
    compiler_params=pltpu.CompilerParams(use_tc_tiling_on_sc=False),
)
def _lookup(idx_hbm, table_hbm, out_hbm, idx_v, rows0, rows1, sg0, sg1, so0, so1):
    wid = lax.axis_index("s") * _NC + lax.axis_index("c")
    out_row0 = wid * _PER_W
    rows = (rows0, rows1)
    sg = (sg0, sg1)
    so = (so0, so1)

    def fire_gather(c, b):
        for j in range(_SUB):
            pltpu.async_copy(
                table_hbm.at[idx_v.at[c * _SUB + j]],
                rows[b].at[pl.ds(j * 128, 128)],
                sg[b],
            )

    def wait_gather(b):
        pltpu.make_async_copy(
            table_hbm.at[pl.ds(0, _CHUNK)], rows[b], sg[b]
        ).wait()

    pltpu.sync_copy(idx_hbm.at[pl.ds(wid * _ROWS_PER_W, _ROWS_PER_W)], idx_v)

    fire_gather(0, 0)
    fire_gather(1, 1)

    def body(g, carry):
        for b in range(2):
            c = 2 * g + b
            wait_gather(b)
            w = pltpu.async_copy(
                rows[b], out_hbm.at[pl.ds(out_row0 + c * _CHUNK, _CHUNK)], so[b]
            )
            w.wait()
            fire_gather(c + 2, b)
        return carry

    lax.fori_loop(0, _NCHUNK // 2 - 1, body, 0)

    for b in range(2):
        c = _NCHUNK - 2 + b
        wait_gather(b)
        pltpu.async_copy(
            rows[b], out_hbm.at[pl.ds(out_row0 + c * _CHUNK, _CHUNK)], so[b]
        ).wait()


def kernel(idxes, pe):
    idx2d = idxes.astype(jnp.int32).reshape(_IDX_ROWS, 128)
    out = _lookup(idx2d, pe)
    return out.reshape(_B, _L, _DIM)


# speedup vs baseline: 4.9776x; 1.0010x over previous
import functools

import jax
import jax.numpy as jnp
from jax import lax
from jax.experimental import pallas as pl
from jax.experimental.pallas import tpu as pltpu
from jax.experimental.pallas import tpu_sc as plsc

_B = 4096
_L = 200
_DIM = 64
_NTOT = _B * _L
_NC = 2
_NS = 16
_NW = _NC * _NS
_PER_W = _NTOT // _NW
_CHUNK = 512
_NCHUNK = _PER_W // _CHUNK
_SUB = _CHUNK // 128
_IDX_ROWS = _NTOT // 128
_ROWS_PER_W = _PER_W // 128


@functools.partial(
    pl.kernel,
    mesh=plsc.VectorSubcoreMesh(core_axis_name="c", subcore_axis_name="s"),
    out_type=jax.ShapeDtypeStruct((_NTOT, _DIM), jnp.float32),
    scratch_types=[
        pltpu.VMEM((_PER_W,), jnp.int32),
        pltpu.VMEM((_CHUNK, _DIM), jnp.float32),
        pltpu.VMEM((_CHUNK, _DIM), jnp.float32),
        pltpu.SemaphoreType.DMA,
        pltpu.SemaphoreType.DMA,
        pltpu.SemaphoreType.DMA,
        pltpu.SemaphoreType.DMA,
    ],
    compiler_params=pltpu.CompilerParams(use_tc_tiling_on_sc=False),
)
def _lookup(idx_hbm, table_hbm, out_hbm, idx_v, rows0, rows1, sg0, sg1, so0, so1):
    wid = lax.axis_index("s") * _NC + lax.axis_index("c")
    out_row0 = wid * _PER_W
    rows = (rows0, rows1)
    sg = (sg0, sg1)
    so = (so0, so1)

    def fire_gather(c, b):
        pltpu.async_copy(
            table_hbm.at[idx_v.at[pl.ds(c * _CHUNK, _CHUNK)]],
            rows[b],
            sg[b],
        )

    def wait_gather(b):
        pltpu.make_async_copy(
            table_hbm.at[pl.ds(0, _CHUNK)], rows[b], sg[b]
        ).wait()

    pltpu.sync_copy(idx_hbm.at[pl.ds(wid * _PER_W, _PER_W)], idx_v)

    fire_gather(0, 0)
    fire_gather(1, 1)

    def body(g, carry):
        for b in range(2):
            c = 2 * g + b
            wait_gather(b)
            w = pltpu.async_copy(
                rows[b], out_hbm.at[pl.ds(out_row0 + c * _CHUNK, _CHUNK)], so[b]
            )
            w.wait()
            fire_gather(c + 2, b)
        return carry

    lax.fori_loop(0, _NCHUNK // 2 - 1, body, 0)

    for b in range(2):
        c = _NCHUNK - 2 + b
        wait_gather(b)
        pltpu.async_copy(
            rows[b], out_hbm.at[pl.ds(out_row0 + c * _CHUNK, _CHUNK)], so[b]
        ).wait()


def kernel(idxes, pe):
    idx_flat = idxes.astype(jnp.int32).reshape(_NTOT)
    out = _lookup(idx_flat, pe)
    return out.reshape(_B, _L, _DIM)
